# baseline (speedup 1.0000x reference)
import functools

import jax
import jax.numpy as jnp
from jax import lax
from jax.experimental import pallas as pl
from jax.experimental.pallas import tpu as pltpu
from jax.experimental.pallas import tpu_sc as plsc

_B = 1048576
_D = 16
_K = 64
_BB = 65536

_NW = 32
_BPW = _B // _NW
_CH = 4096


def _vq_block(at_ref, ct_ref, c2_ref, g_ref, idx_ref, msum_ref):
    at = at_ref[...]
    e = jax.lax.dot_general(ct_ref[...], at, (((1,), (0,)), ((), ())),
                            preferred_element_type=jnp.float32)
    e = e + c2_ref[...]
    m = jnp.min(e, axis=0, keepdims=True)
    onehot = (e == m).astype(jnp.float32)
    out2 = jax.lax.dot_general(g_ref[...], onehot, (((1,), (0,)), ((), ())),
                               preferred_element_type=jnp.float32)
    idx_ref[...] = out2.astype(jnp.int32)
    s_val = jnp.sum(at * at) + jnp.sum(m)

    @pl.when(pl.program_id(0) == 0)
    def _init():
        msum_ref[0, 0] = 0.0

    msum_ref[0, 0] += s_val


def _tc_stage(at, ct, c2, g):
    n_blocks = _B // _BB
    return pl.pallas_call(
        _vq_block,
        grid=(n_blocks,),
        in_specs=[
            pl.BlockSpec((_D, _BB), lambda i: (0, i)),
            pl.BlockSpec((_K, _D), lambda i: (0, 0)),
            pl.BlockSpec((_K, 1), lambda i: (0, 0)),
            pl.BlockSpec((1, _K), lambda i: (0, 0)),
        ],
        out_specs=[
            pl.BlockSpec((1, _BB), lambda i: (0, i)),
            pl.BlockSpec((1, 1), lambda i: (0, 0), memory_space=pltpu.SMEM),
        ],
        out_shape=[
            jax.ShapeDtypeStruct((1, _B), jnp.int32),
            jax.ShapeDtypeStruct((1, 1), jnp.float32),
        ],
    )(at, ct, c2, g)


def _sc_gather(ctt_flat, idx):
    mesh = plsc.VectorSubcoreMesh(core_axis_name="c", subcore_axis_name="s")
    info = plsc.get_sparse_core_info()
    nc = info.num_cores

    @functools.partial(
        pl.kernel, mesh=mesh,
        out_type=jax.ShapeDtypeStruct((_D, _B), jnp.float32),
        compiler_params=pltpu.CompilerParams(needs_layout_passes=False),
        scratch_types=[
            pltpu.VMEM((_D * _K,), jnp.float32),
            pltpu.VMEM((_CH,), jnp.int32),
            pltpu.VMEM((_D * _CH,), jnp.float32),
        ],
    )
    def k(ct_hbm, idx_hbm, out_hbm, ct_v, idx_v, buf_v):
        wid = lax.axis_index("s") * nc + lax.axis_index("c")
        pltpu.sync_copy(ct_hbm, ct_v)
        base = wid * _BPW
        for chunk in range(_BPW // _CH):
            cb = base + chunk * _CH
            pltpu.sync_copy(idx_hbm.at[0, pl.ds(cb, _CH)], idx_v)

            def body(i, carry):
                idxv = idx_v[pl.ds(i * 16, 16)]
                for d in range(_D):
                    buf_v[pl.ds(d * _CH + i * 16, 16)] = plsc.load_gather(
                        ct_v, [idxv + _K * d])
                return carry

            lax.fori_loop(0, _CH // 16, body, 0)
            for d in range(_D):
                pltpu.sync_copy(buf_v.at[pl.ds(d * _CH, _CH)],
                                out_hbm.at[d, pl.ds(cb, _CH)])

    return k(ctt_flat, idx)


def kernel(action, codebook):
    at = action.T
    ct = -2.0 * codebook
    c2 = jnp.sum(codebook * codebook, axis=1)[:, None]
    g = jnp.arange(_K, dtype=jnp.float32)[None, :]
    idx, msum = _tc_stage(at, ct, c2, g)
    ctt_flat = codebook.T.reshape(_D * _K)
    zqt = _sc_gather(ctt_flat, idx)
    mean_dist = msum[0, 0] / _B
    return (zqt.T, idx.reshape(_B), mean_dist)


# speedup vs baseline: 1.0204x; 1.0204x over previous
import functools

import jax
import jax.numpy as jnp
from jax import lax
from jax.experimental import pallas as pl
from jax.experimental.pallas import tpu as pltpu
from jax.experimental.pallas import tpu_sc as plsc

_B = 1048576
_D = 16
_K = 64
_BB = 65536

_NW = 32
_BPW = _B // _NW
_CH = 2048


def _vq_block(at_ref, ct_ref, c2_ref, g_ref, idx_ref, msum_ref):
    at = at_ref[...]
    e = jax.lax.dot_general(ct_ref[...], at, (((1,), (0,)), ((), ())),
                            preferred_element_type=jnp.float32)
    e = e + c2_ref[...]
    m = jnp.min(e, axis=0, keepdims=True)
    onehot = (e == m).astype(jnp.float32)
    out2 = jax.lax.dot_general(g_ref[...], onehot, (((1,), (0,)), ((), ())),
                               preferred_element_type=jnp.float32)
    idx_ref[...] = out2.astype(jnp.int32)
    s_val = jnp.sum(at * at) + jnp.sum(m)

    @pl.when(pl.program_id(0) == 0)
    def _init():
        msum_ref[0, 0] = 0.0

    msum_ref[0, 0] += s_val


def _tc_stage(at, ct, c2, g):
    n_blocks = _B // _BB
    return pl.pallas_call(
        _vq_block,
        grid=(n_blocks,),
        in_specs=[
            pl.BlockSpec((_D, _BB), lambda i: (0, i)),
            pl.BlockSpec((_K, _D), lambda i: (0, 0)),
            pl.BlockSpec((_K, 1), lambda i: (0, 0)),
            pl.BlockSpec((1, _K), lambda i: (0, 0)),
        ],
        out_specs=[
            pl.BlockSpec((1, _BB), lambda i: (0, i)),
            pl.BlockSpec((1, 1), lambda i: (0, 0), memory_space=pltpu.SMEM),
        ],
        out_shape=[
            jax.ShapeDtypeStruct((1, _B), jnp.int32),
            jax.ShapeDtypeStruct((1, 1), jnp.float32),
        ],
    )(at, ct, c2, g)


def _sc_gather(ctt_flat, idx):
    mesh = plsc.VectorSubcoreMesh(core_axis_name="c", subcore_axis_name="s")
    info = plsc.get_sparse_core_info()
    nc = info.num_cores

    @functools.partial(
        pl.kernel, mesh=mesh,
        out_type=jax.ShapeDtypeStruct((_D, _B), jnp.float32),
        compiler_params=pltpu.CompilerParams(needs_layout_passes=False),
        scratch_types=[
            pltpu.VMEM((_D * _K,), jnp.float32),
            pltpu.VMEM((2, _CH), jnp.int32),
            pltpu.VMEM((2, _D * _CH), jnp.float32),
            pltpu.SemaphoreType.DMA,
        ],
    )
    def k(ct_hbm, idx_hbm, out_hbm, ct_v, idx_v, buf_v, sem_o):
        wid = lax.axis_index("s") * nc + lax.axis_index("c")
        pltpu.sync_copy(ct_hbm, ct_v)
        base = wid * _BPW
        nch = _BPW // _CH

        def pair(j, carry):
            for b in (0, 1):
                cb = base + (2 * j + b) * _CH
                pltpu.sync_copy(idx_hbm.at[0, pl.ds(cb, _CH)], idx_v.at[b])

                @pl.when(j > 0)
                def _drain(b=b, cb=cb):
                    for d in range(_D):
                        pltpu.make_async_copy(
                            buf_v.at[b, pl.ds(d * _CH, _CH)],
                            out_hbm.at[d, pl.ds(cb, _CH)], sem_o).wait()

                def body(i, c, b=b):
                    idxv = idx_v[b, pl.ds(i * 16, 16)]
                    for d in range(_D):
                        buf_v[b, pl.ds(d * _CH + i * 16, 16)] = (
                            plsc.load_gather(ct_v, [idxv + _K * d]))
                    return c

                lax.fori_loop(0, _CH // 16, body, 0, unroll=4)
                for d in range(_D):
                    pltpu.async_copy(buf_v.at[b, pl.ds(d * _CH, _CH)],
                                     out_hbm.at[d, pl.ds(cb, _CH)], sem_o)
            return carry

        lax.fori_loop(0, nch // 2, pair, 0)
        for b in (0, 1):
            for d in range(_D):
                pltpu.make_async_copy(
                    buf_v.at[b, pl.ds(d * _CH, _CH)],
                    out_hbm.at[d, pl.ds(base, _CH)], sem_o).wait()

    return k(ctt_flat, idx)


def kernel(action, codebook):
    at = action.T
    ct = -2.0 * codebook
    c2 = jnp.sum(codebook * codebook, axis=1)[:, None]
    g = jnp.arange(_K, dtype=jnp.float32)[None, :]
    idx, msum = _tc_stage(at, ct, c2, g)
    ctt_flat = codebook.T.reshape(_D * _K)
    zqt = _sc_gather(ctt_flat, idx)
    mean_dist = msum[0, 0] / _B
    return (zqt.T, idx.reshape(_B), mean_dist)


# speedup vs baseline: 3.7517x; 3.6766x over previous
import functools

import jax
import jax.numpy as jnp
from jax.experimental import pallas as pl
from jax.experimental.pallas import tpu as pltpu

_B = 1048576
_D = 16
_K = 64
_BB = 65536


def _vq_block(at_ref, ct_ref, c2_ref, g_ref, zqt_ref, idx_ref, msum_ref):
    at = at_ref[...]
    e = jax.lax.dot_general(ct_ref[...], at, (((1,), (0,)), ((), ())),
                            preferred_element_type=jnp.float32)
    e = e + c2_ref[...]
    m = jnp.min(e, axis=0, keepdims=True)
    onehot = (e == m).astype(jnp.float32)
    out2 = jax.lax.dot_general(g_ref[...], onehot, (((1,), (0,)), ((), ())),
                               preferred_element_type=jnp.float32)
    zqt_ref[...] = out2[:_D, :]
    idx_ref[...] = out2[_D:_D + 1, :].astype(jnp.int32)
    s_val = jnp.sum(at * at) + jnp.sum(m)

    @pl.when(pl.program_id(0) == 0)
    def _init():
        msum_ref[0, 0] = 0.0

    msum_ref[0, 0] += s_val


def kernel(action, codebook):
    at = action.T
    ct = -2.0 * codebook
    c2 = jnp.sum(codebook * codebook, axis=1)[:, None]
    kf = jnp.arange(_K, dtype=jnp.float32)[None, :]
    g = jnp.concatenate([codebook.T, kf], axis=0)
    n_blocks = _B // _BB
    zqt, idx, msum = pl.pallas_call(
        _vq_block,
        grid=(n_blocks,),
        in_specs=[
            pl.BlockSpec((_D, _BB), lambda i: (0, i)),
            pl.BlockSpec((_K, _D), lambda i: (0, 0)),
            pl.BlockSpec((_K, 1), lambda i: (0, 0)),
            pl.BlockSpec((_D + 1, _K), lambda i: (0, 0)),
        ],
        out_specs=[
            pl.BlockSpec((_D, _BB), lambda i: (0, i)),
            pl.BlockSpec((1, _BB), lambda i: (0, i)),
            pl.BlockSpec((1, 1), lambda i: (0, 0), memory_space=pltpu.SMEM),
        ],
        out_shape=[
            jax.ShapeDtypeStruct((_D, _B), jnp.float32),
            jax.ShapeDtypeStruct((1, _B), jnp.int32),
            jax.ShapeDtypeStruct((1, 1), jnp.float32),
        ],
    )(at, ct, c2, g)
    mean_dist = msum[0, 0] / _B
    return (zqt.T, idx.reshape(_B), mean_dist)
